# baseline (speedup 1.0000x reference)
import jax
import jax.numpy as jnp
from jax import lax
from jax.experimental import pallas as pl
from jax.experimental.pallas import tpu as pltpu
from jax.experimental.pallas import tpu_sc as plsc

DIM = 768
NC = 2
NS = 16
NW = NC * NS
LC = 50
LQ = 20
LCP = 56
LQP = 24


def _gather_body(ctx_idx, q_idx, table, ctx_out, q_out,
                 idxc_v, idxq_v, bufc0, bufc1, bufq0, bufq1, sem0, sem1):
    wid = lax.axis_index("s") * NC + lax.axis_index("c")
    nb = ctx_idx.shape[0] // (NW * LCP)
    base = wid * nb

    pltpu.sync_copy(ctx_idx.at[pl.ds(base * LCP, nb * LCP)], idxc_v)
    pltpu.sync_copy(q_idx.at[pl.ds(base * LQP, nb * LQP)], idxq_v)

    def phase(idx_v, out_hbm, bufs, rows, stride):
        sems = (sem0, sem1)

        def start(j, p):
            pltpu.make_async_copy(
                table.at[idx_v.at[pl.ds(j * stride, rows)]],
                bufs[p].at[0], sems[p]).start()

        def wait(p):
            pltpu.make_async_copy(
                table.at[idx_v.at[pl.ds(0, rows)]], bufs[p].at[0],
                sems[p]).wait()

        start(0, 0)
        start(1, 1)

        @pl.loop(0, nb // 2)
        def _pair(i):
            for p in (0, 1):
                j = 2 * i + p
                wait(p)
                pltpu.sync_copy(bufs[p], out_hbm.at[pl.ds(base + j, 1)])

                @pl.when(j + 2 < nb)
                def _():
                    start(j + 2, p)

    phase(idxc_v, ctx_out, (bufc0, bufc1), LCP, LCP)
    phase(idxq_v, q_out, (bufq0, bufq1), LQP, LQP)


@jax.jit
def _gather(ctx_idx, q_idx, table):
    b = ctx_idx.shape[0] // LCP
    mesh = plsc.VectorSubcoreMesh(core_axis_name="c", subcore_axis_name="s")
    f = pl.kernel(
        _gather_body,
        out_type=(
            jax.ShapeDtypeStruct((b, LCP, DIM), jnp.float32),
            jax.ShapeDtypeStruct((b, LQP, DIM), jnp.float32),
        ),
        mesh=mesh,
        scratch_types=[
            pltpu.VMEM((b // NW * LCP,), jnp.int32),
            pltpu.VMEM((b // NW * LQP,), jnp.int32),
            pltpu.VMEM((1, LCP, DIM), jnp.float32),
            pltpu.VMEM((1, LCP, DIM), jnp.float32),
            pltpu.VMEM((1, LQP, DIM), jnp.float32),
            pltpu.VMEM((1, LQP, DIM), jnp.float32),
            pltpu.SemaphoreType.DMA,
            pltpu.SemaphoreType.DMA,
        ],
    )
    return f(ctx_idx, q_idx, table)


def _pad_flat(idx, stride):
    pad = jnp.zeros((idx.shape[0], stride - idx.shape[1]), jnp.int32)
    return jnp.concatenate([idx.astype(jnp.int32), pad], axis=1).reshape(-1)


def kernel(context, context_lengths, question, question_lengths, table):
    ctx_p, q_p = _gather(_pad_flat(context, LCP), _pad_flat(question, LQP),
                         table)
    ctx_e = ctx_p[:, :LC, :]
    q_e = q_p[:, :LQ, :]
    return (ctx_e, ctx_e, q_e)


# speedup vs baseline: 1.7182x; 1.7182x over previous
import jax
import jax.numpy as jnp
from jax import lax
from jax.experimental import pallas as pl
from jax.experimental.pallas import tpu as pltpu
from jax.experimental.pallas import tpu_sc as plsc

DIM = 768
NC = 2
NS = 16
NW = NC * NS
LC = 50
LQ = 20
K = 80
G = 8


def _gather_body(ctx_idx, q_idx, table, ctx_out, q_out,
                 idx_v, buf0, buf1, sem0, sem1):
    wid = lax.axis_index("s") * NC + lax.axis_index("c")
    ctx_per_w = ctx_idx.shape[0] // NW
    q_per_w = q_idx.shape[0] // NW
    n_ctx_chunks = ctx_per_w // K
    n_chunks = (ctx_per_w + q_per_w) // K

    pltpu.sync_copy(ctx_idx.at[pl.ds(wid * ctx_per_w, ctx_per_w)],
                    idx_v.at[pl.ds(0, ctx_per_w)])
    pltpu.sync_copy(q_idx.at[pl.ds(wid * q_per_w, q_per_w)],
                    idx_v.at[pl.ds(ctx_per_w, q_per_w)])

    def start_gather(c, buf, sem):
        pltpu.make_async_copy(
            table.at[idx_v.at[pl.ds(c * K, K)]], buf, sem).start()

    def wait_gather(buf, sem):
        pltpu.make_async_copy(
            table.at[idx_v.at[pl.ds(0, K)]], buf, sem).wait()

    def store_chunk(c, buf):
        @pl.when(c < n_ctx_chunks)
        def _():
            pltpu.sync_copy(
                buf, ctx_out.at[pl.ds(wid * ctx_per_w + c * K, K)])

        @pl.when(c >= n_ctx_chunks)
        def _():
            pltpu.sync_copy(
                buf, q_out.at[pl.ds(wid * q_per_w + (c - n_ctx_chunks) * K, K)])

    start_gather(0, buf0, sem0)
    start_gather(1, buf1, sem1)

    @pl.loop(0, n_chunks // 2)
    def _pair(j):
        for buf, sem, par in ((buf0, sem0, 0), (buf1, sem1, 1)):
            c = 2 * j + par
            wait_gather(buf, sem)
            store_chunk(c, buf)

            @pl.when(c + 2 < n_chunks)
            def _():
                start_gather(c + 2, buf, sem)


def _reshape_body(ctx_ref, q_ref, ca_ref, cb_ref, q3_ref):
    for p in range(G):
        page = ctx_ref[pl.ds(p * LC, LC), :]
        ca_ref[p] = page
        cb_ref[p] = page
        q3_ref[p] = q_ref[pl.ds(p * LQ, LQ), :]


@jax.jit
def _run(ctx_idx, q_idx, table):
    n_ctx = ctx_idx.shape[0]
    n_q = q_idx.shape[0]
    b = n_ctx // LC
    mesh = plsc.VectorSubcoreMesh(core_axis_name="c", subcore_axis_name="s")
    gather = pl.kernel(
        _gather_body,
        out_type=(
            jax.ShapeDtypeStruct((n_ctx, DIM), jnp.float32),
            jax.ShapeDtypeStruct((n_q, DIM), jnp.float32),
        ),
        mesh=mesh,
        scratch_types=[
            pltpu.VMEM(((n_ctx + n_q) // NW,), jnp.int32),
            pltpu.VMEM((K, DIM), jnp.float32),
            pltpu.VMEM((K, DIM), jnp.float32),
            pltpu.SemaphoreType.DMA,
            pltpu.SemaphoreType.DMA,
        ],
    )
    ctx2, q2 = gather(ctx_idx, q_idx, table)

    reshape = pl.pallas_call(
        _reshape_body,
        grid=(b // G,),
        in_specs=[
            pl.BlockSpec((G * LC, DIM), lambda g: (g, 0)),
            pl.BlockSpec((G * LQ, DIM), lambda g: (g, 0)),
        ],
        out_specs=[
            pl.BlockSpec((G, LC, DIM), lambda g: (g, 0, 0)),
            pl.BlockSpec((G, LC, DIM), lambda g: (g, 0, 0)),
            pl.BlockSpec((G, LQ, DIM), lambda g: (g, 0, 0)),
        ],
        out_shape=[
            jax.ShapeDtypeStruct((b, LC, DIM), jnp.float32),
            jax.ShapeDtypeStruct((b, LC, DIM), jnp.float32),
            jax.ShapeDtypeStruct((b, LQ, DIM), jnp.float32),
        ],
    )
    return reshape(ctx2, q2)


def kernel(context, context_lengths, question, question_lengths, table):
    ctx_idx = context.reshape(-1).astype(jnp.int32)
    q_idx = question.reshape(-1).astype(jnp.int32)
    ca, cb, q3 = _run(ctx_idx, q_idx, table)
    return (ca, cb, q3)


# speedup vs baseline: 4.0961x; 2.3839x over previous
import jax
import jax.numpy as jnp
from jax import lax
from jax.experimental import pallas as pl
from jax.experimental.pallas import tpu as pltpu
from jax.experimental.pallas import tpu_sc as plsc

DIM = 768
NC = 2
NS = 16
NW = NC * NS
LC = 50
LQ = 20
K = 64


def _gather_body(idx_all, table, ctx_out, q_out, idx_v, buf0, buf1,
                 sem0, sem1):
    wid = lax.axis_index("s") * NC + lax.axis_index("c")
    b = ctx_out.shape[1]
    kpb = b // K
    n_ctx_chunks = LC * kpb
    n_chunks = (LC + LQ) * kpb
    per_w = n_chunks // NW
    rows_w = per_w * K
    c0 = wid * per_w

    pltpu.sync_copy(idx_all.at[pl.ds(wid * rows_w, rows_w)], idx_v)

    def start_gather(j, buf, sem):
        pltpu.make_async_copy(
            table.at[idx_v.at[pl.ds(j * K, K)]], buf, sem).start()

    def wait_gather(buf, sem):
        pltpu.make_async_copy(
            table.at[idx_v.at[pl.ds(0, K)]], buf, sem).wait()

    def store_chunk(j, buf):
        c = c0 + j

        @pl.when(c < n_ctx_chunks)
        def _():
            pltpu.sync_copy(
                buf, ctx_out.at[c // kpb, pl.ds((c % kpb) * K, K)])

        @pl.when(c >= n_ctx_chunks)
        def _():
            c2 = c - n_ctx_chunks
            pltpu.sync_copy(
                buf, q_out.at[c2 // kpb, pl.ds((c2 % kpb) * K, K)])

    start_gather(0, buf0, sem0)
    start_gather(1, buf1, sem1)

    @pl.loop(0, per_w // 2)
    def _pair(i):
        for buf, sem, par in ((buf0, sem0, 0), (buf1, sem1, 1)):
            j = 2 * i + par
            wait_gather(buf, sem)
            store_chunk(j, buf)

            @pl.when(j + 2 < per_w)
            def _():
                start_gather(j + 2, buf, sem)

    if per_w % 2:
        wait_gather(buf0, sem0)
        store_chunk(per_w - 1, buf0)


@jax.jit
def _gather(idx_all, table):
    b = 1024
    mesh = plsc.VectorSubcoreMesh(core_axis_name="c", subcore_axis_name="s")
    f = pl.kernel(
        _gather_body,
        out_type=(
            jax.ShapeDtypeStruct((LC, b, DIM), jnp.float32),
            jax.ShapeDtypeStruct((LQ, b, DIM), jnp.float32),
        ),
        mesh=mesh,
        scratch_types=[
            pltpu.VMEM((idx_all.shape[0] // NW,), jnp.int32),
            pltpu.VMEM((K, DIM), jnp.float32),
            pltpu.VMEM((K, DIM), jnp.float32),
            pltpu.SemaphoreType.DMA,
            pltpu.SemaphoreType.DMA,
        ],
    )
    return f(idx_all, table)


def kernel(context, context_lengths, question, question_lengths, table):
    idx_all = jnp.concatenate(
        [context.T.reshape(-1), question.T.reshape(-1)]).astype(jnp.int32)
    ctx_t, q_t = _gather(idx_all, table)
    ctx_e = jnp.transpose(ctx_t, (1, 0, 2))
    q_e = jnp.transpose(q_t, (1, 0, 2))
    return (ctx_e, ctx_e, q_e)


# speedup vs baseline: 4.6216x; 1.1283x over previous
import jax
import jax.numpy as jnp
from jax import lax
from jax.experimental import pallas as pl
from jax.experimental.pallas import tpu as pltpu
from jax.experimental.pallas import tpu_sc as plsc

DIM = 768
NC = 2
NS = 16
NW = NC * NS
LC = 50
LQ = 20
K = 64


def _gather_body(idx_all, table, ctx_out, ctx_out2, q_out, idx_v, buf0, buf1,
                 sem0, sem1):
    wid = lax.axis_index("s") * NC + lax.axis_index("c")
    b = ctx_out.shape[1]
    kpb = b // K
    n_ctx_chunks = LC * kpb
    n_chunks = (LC + LQ) * kpb
    per_w = n_chunks // NW
    rows_w = per_w * K
    c0 = wid * per_w

    pltpu.sync_copy(idx_all.at[pl.ds(wid * rows_w, rows_w)], idx_v)

    def start_gather(j, buf, sem):
        pltpu.make_async_copy(
            table.at[idx_v.at[pl.ds(j * K, K)]], buf, sem).start()

    def wait_gather(buf, sem):
        pltpu.make_async_copy(
            table.at[idx_v.at[pl.ds(0, K)]], buf, sem).wait()

    def store_chunk(j, buf):
        c = c0 + j

        @pl.when(c < n_ctx_chunks)
        def _():
            pltpu.sync_copy(
                buf, ctx_out.at[c // kpb, pl.ds((c % kpb) * K, K)])
            pltpu.sync_copy(
                buf, ctx_out2.at[c // kpb, pl.ds((c % kpb) * K, K)])

        @pl.when(c >= n_ctx_chunks)
        def _():
            c2 = c - n_ctx_chunks
            pltpu.sync_copy(
                buf, q_out.at[c2 // kpb, pl.ds((c2 % kpb) * K, K)])

    start_gather(0, buf0, sem0)
    start_gather(1, buf1, sem1)

    @pl.loop(0, per_w // 2)
    def _pair(i):
        for buf, sem, par in ((buf0, sem0, 0), (buf1, sem1, 1)):
            j = 2 * i + par
            wait_gather(buf, sem)
            store_chunk(j, buf)

            @pl.when(j + 2 < per_w)
            def _():
                start_gather(j + 2, buf, sem)

    if per_w % 2:
        wait_gather(buf0, sem0)
        store_chunk(per_w - 1, buf0)


@jax.jit
def _gather(idx_all, table):
    b = 1024
    mesh = plsc.VectorSubcoreMesh(core_axis_name="c", subcore_axis_name="s")
    f = pl.kernel(
        _gather_body,
        out_type=(
            jax.ShapeDtypeStruct((LC, b, DIM), jnp.float32),
            jax.ShapeDtypeStruct((LC, b, DIM), jnp.float32),
            jax.ShapeDtypeStruct((LQ, b, DIM), jnp.float32),
        ),
        mesh=mesh,
        scratch_types=[
            pltpu.VMEM((idx_all.shape[0] // NW,), jnp.int32),
            pltpu.VMEM((K, DIM), jnp.float32),
            pltpu.VMEM((K, DIM), jnp.float32),
            pltpu.SemaphoreType.DMA,
            pltpu.SemaphoreType.DMA,
        ],
    )
    return f(idx_all, table)


def kernel(context, context_lengths, question, question_lengths, table):
    idx_all = jnp.concatenate(
        [context.T.reshape(-1), question.T.reshape(-1)]).astype(jnp.int32)
    ctx_t, ctx_t2, q_t = _gather(idx_all, table)
    ctx_e = jnp.transpose(ctx_t, (1, 0, 2))
    ctx_e2 = jnp.transpose(ctx_t2, (1, 0, 2))
    q_e = jnp.transpose(q_t, (1, 0, 2))
    return (ctx_e, ctx_e2, q_e)


# speedup vs baseline: 4.7539x; 1.0286x over previous
import jax
import jax.numpy as jnp
from jax import lax
from jax.experimental import pallas as pl
from jax.experimental.pallas import tpu as pltpu
from jax.experimental.pallas import tpu_sc as plsc

DIM = 768
NC = 2
NS = 16
NW = NC * NS
LC = 50
LQ = 20
K = 64


def _gather_body(idx_all, table, ctx_out, ctx_out2, q_out, idx_v, buf0, buf1,
                 sem0, sem1, ssem0, ssem1):
    wid = lax.axis_index("s") * NC + lax.axis_index("c")
    b = ctx_out.shape[1]
    kpb = b // K
    n_ctx_chunks = LC * kpb
    n_chunks = (LC + LQ) * kpb
    per_w = n_chunks // NW
    rows_w = per_w * K

    pltpu.sync_copy(idx_all.at[pl.ds(wid * rows_w, rows_w)], idx_v)

    def start_gather(j, buf, sem):
        pltpu.make_async_copy(
            table.at[idx_v.at[pl.ds(j * K, K)]], buf, sem).start()

    def wait_gather(buf, sem):
        pltpu.make_async_copy(
            table.at[idx_v.at[pl.ds(0, K)]], buf, sem).wait()

    def store_chunk(j, buf, ssem, ssem2):
        c = j * NW + wid

        @pl.when(c < n_ctx_chunks)
        def _():
            d1 = ctx_out.at[c // kpb, pl.ds((c % kpb) * K, K)]
            d2 = ctx_out2.at[c // kpb, pl.ds((c % kpb) * K, K)]
            cp1 = pltpu.make_async_copy(buf, d1, ssem)
            cp2 = pltpu.make_async_copy(buf, d2, ssem2)
            cp1.start()
            cp2.start()
            cp1.wait()
            cp2.wait()

        @pl.when(c >= n_ctx_chunks)
        def _():
            c2 = c - n_ctx_chunks
            pltpu.sync_copy(
                buf, q_out.at[c2 // kpb, pl.ds((c2 % kpb) * K, K)])

    start_gather(0, buf0, sem0)
    start_gather(1, buf1, sem1)

    @pl.loop(0, per_w // 2)
    def _pair(i):
        for buf, sem, par in ((buf0, sem0, 0), (buf1, sem1, 1)):
            j = 2 * i + par
            wait_gather(buf, sem)
            store_chunk(j, buf, ssem0, ssem1)

            @pl.when(j + 2 < per_w)
            def _():
                start_gather(j + 2, buf, sem)

    if per_w % 2:
        wait_gather(buf0, sem0)
        store_chunk(per_w - 1, buf0, ssem0, ssem1)


@jax.jit
def _gather(idx_all, table):
    b = 1024
    mesh = plsc.VectorSubcoreMesh(core_axis_name="c", subcore_axis_name="s")
    f = pl.kernel(
        _gather_body,
        out_type=(
            jax.ShapeDtypeStruct((LC, b, DIM), jnp.float32),
            jax.ShapeDtypeStruct((LC, b, DIM), jnp.float32),
            jax.ShapeDtypeStruct((LQ, b, DIM), jnp.float32),
        ),
        mesh=mesh,
        scratch_types=[
            pltpu.VMEM((idx_all.shape[0] // NW,), jnp.int32),
            pltpu.VMEM((K, DIM), jnp.float32),
            pltpu.VMEM((K, DIM), jnp.float32),
            pltpu.SemaphoreType.DMA,
            pltpu.SemaphoreType.DMA,
            pltpu.SemaphoreType.DMA,
            pltpu.SemaphoreType.DMA,
        ],
    )
    return f(idx_all, table)


def kernel(context, context_lengths, question, question_lengths, table):
    idx_flat = jnp.concatenate(
        [context.T.reshape(-1), question.T.reshape(-1)]).astype(jnp.int32)
    n_chunks = idx_flat.shape[0] // K
    idx_all = jnp.transpose(
        idx_flat.reshape(n_chunks // NW, NW, K), (1, 0, 2)).reshape(-1)
    ctx_t, ctx_t2, q_t = _gather(idx_all, table)
    ctx_e = jnp.transpose(ctx_t, (1, 0, 2))
    ctx_e2 = jnp.transpose(ctx_t2, (1, 0, 2))
    q_e = jnp.transpose(q_t, (1, 0, 2))
    return (ctx_e, ctx_e2, q_e)
